# Initial kernel scaffold; baseline (speedup 1.0000x reference)
import jax
import jax.numpy as jnp
from jax.experimental import pallas as pl


def kernel(anchors, scores):
    raise NotImplementedError("write your pallas kernel here")


# speedup vs baseline: 35.0244x; 35.0244x over previous
import functools

import jax
import jax.numpy as jnp
from jax.experimental import pallas as pl
from jax.experimental.pallas import tpu as pltpu

IOU_THRESHOLD = 0.6
NMS_FILTER = 2000
BLK = 128
PAD_COORD = -1.0e6


def _nms_block_kernel(x1_ref, y1_ref, x2_ref, y2_ref,
                      keep_ref, removed_ref, m_ref, nblk: int):
    b = pl.program_id(0)

    @pl.when(b == 0)
    def _init():
        removed_ref[...] = jnp.zeros_like(removed_ref)

    cx1 = x1_ref[pl.ds(b, 1), :]
    cy1 = y1_ref[pl.ds(b, 1), :]
    cx2 = x2_ref[pl.ds(b, 1), :]
    cy2 = y2_ref[pl.ds(b, 1), :]
    rx1 = jnp.broadcast_to(cx1, (BLK, BLK)).T
    ry1 = jnp.broadcast_to(cy1, (BLK, BLK)).T
    rx2 = jnp.broadcast_to(cx2, (BLK, BLK)).T
    ry2 = jnp.broadcast_to(cy2, (BLK, BLK)).T
    rarea = (rx2 - rx1) * (ry2 - ry1)

    def iou_gt(cx1, cy1, cx2, cy2):
        ix1 = jnp.maximum(rx1, cx1)
        iy1 = jnp.maximum(ry1, cy1)
        ix2 = jnp.minimum(rx2, cx2)
        iy2 = jnp.minimum(ry2, cy2)
        inter = jnp.maximum(ix2 - ix1, 0.0) * jnp.maximum(iy2 - iy1, 0.0)
        carea = (cx2 - cx1) * (cy2 - cy1)
        iou = inter / (rarea + carea - inter + 1e-9)
        return (iou > IOU_THRESHOLD).astype(jnp.float32)

    m = iou_gt(cx1, cy1, cx2, cy2)
    col_ids = jax.lax.broadcasted_iota(jnp.int32, (BLK, BLK), 1)
    row_ids = jax.lax.broadcasted_iota(jnp.int32, (BLK, BLK), 0)
    m_ref[...] = m * (col_ids > row_ids).astype(jnp.float32)

    lane = jax.lax.broadcasted_iota(jnp.int32, (1, BLK), 1)
    keep0 = 1.0 - removed_ref[pl.ds(b, 1), :]

    def body(i, keep):
        mrow = m_ref[pl.ds(i, 1), :]
        keep_i = jnp.sum(jnp.where(lane == i, keep, 0.0))
        return keep * (1.0 - mrow * keep_i)

    keep = jax.lax.fori_loop(0, BLK, body, keep0)
    keep_ref[pl.ds(b, 1), :] = keep

    keep_rows = jnp.broadcast_to(keep, (BLK, BLK)).T

    def cross(c, _):
        ccx1 = x1_ref[pl.ds(c, 1), :]
        ccy1 = y1_ref[pl.ds(c, 1), :]
        ccx2 = x2_ref[pl.ds(c, 1), :]
        ccy2 = y2_ref[pl.ds(c, 1), :]
        mc = iou_gt(ccx1, ccy1, ccx2, ccy2) * keep_rows
        sup = jnp.max(mc, axis=0, keepdims=True)
        removed_ref[pl.ds(c, 1), :] = jnp.maximum(
            removed_ref[pl.ds(c, 1), :], sup)
        return 0

    jax.lax.fori_loop(b + 1, nblk, cross, 0)


def _run_nms(bs, n_pad, interpret=False):
    nblk = n_pad // BLK
    planes = [bs[:, i].reshape(nblk, BLK) for i in range(4)]
    keep = pl.pallas_call(
        functools.partial(_nms_block_kernel, nblk=nblk),
        grid=(nblk,),
        in_specs=[pl.BlockSpec((nblk, BLK), lambda b: (0, 0))] * 4,
        out_specs=pl.BlockSpec((nblk, BLK), lambda b: (0, 0)),
        out_shape=jax.ShapeDtypeStruct((nblk, BLK), jnp.float32),
        scratch_shapes=[pltpu.VMEM((nblk, BLK), jnp.float32),
                        pltpu.VMEM((BLK, BLK), jnp.float32)],
        interpret=interpret,
    )(*planes)
    return keep.reshape(-1)


def kernel(anchors, scores):
    n = anchors.shape[0]
    n_pad = ((n + BLK - 1) // BLK) * BLK

    scores_fg = scores.reshape(-1, 2)[:, 1]
    top_scores_idx = jnp.argsort(scores_fg)
    top_anchors = anchors[top_scores_idx]
    top_scores = top_anchors[:, 1]
    order = jnp.argsort(-top_scores)
    bs = top_anchors[order]
    bs_pad = jnp.concatenate(
        [bs, jnp.full((n_pad - n, 4), PAD_COORD, jnp.float32)], axis=0)

    keep = _run_nms(bs_pad, n_pad)[:n] > 0.5

    sel = jnp.nonzero(keep, size=min(NMS_FILTER, n), fill_value=0)[0]
    nms_idx = order[sel]
    return anchors[nms_idx], nms_idx
